# baseline (speedup 1.0000x reference)
import functools

import jax
import jax.numpy as jnp
from jax import lax
from jax.experimental import pallas as pl
from jax.experimental.pallas import tpu as pltpu
from jax.experimental.pallas import tpu_sc as plsc

B = 32
S = 32
DIM = 512
HID = 512
G4 = 4 * HID
N_ROWS = B * S
VOCAB = 10000

def _mm(x, w):
    return jnp.dot(x.astype(jnp.bfloat16), w,
                   preferred_element_type=jnp.float32)


@functools.lru_cache(maxsize=1)
def _make_sc_gather():
    info = plsc.get_sparse_core_info()
    nc, ns = info.num_cores, info.num_subcores
    nw = nc * ns
    rows_per_w = N_ROWS // nw
    mesh = plsc.VectorSubcoreMesh(core_axis_name="c", subcore_axis_name="s")

    @functools.partial(
        pl.kernel,
        mesh=mesh,
        out_type=jax.ShapeDtypeStruct((N_ROWS, DIM), jnp.float32),
        scratch_types=[
            pltpu.VMEM((rows_per_w,), jnp.int32),
            pltpu.VMEM((rows_per_w, DIM), jnp.float32),
            pltpu.SemaphoreType.DMA,
        ],
    )
    def gather_k(idx_hbm, table_hbm, out_hbm, idx_v, rows_v, sem):
        wid = lax.axis_index("s") * nc + lax.axis_index("c")
        base = wid * rows_per_w
        pltpu.sync_copy(idx_hbm.at[pl.ds(base, rows_per_w)], idx_v)
        pltpu.async_copy(table_hbm.at[idx_v], rows_v, sem).wait()
        pltpu.sync_copy(rows_v, out_hbm.at[pl.ds(base, rows_per_w)])

    return gather_k


def _lstm_body(x_ref, wih0T_ref, whh0T_ref, wcat1T_ref, b0_ref, b1_ref,
               y_ref, xi0_ref, hcat_ref, c1_ref, c2_ref):
    xi0_ref[...] = _mm(x_ref[...], wih0T_ref[...]) + b0_ref[...]
    hcat_ref[...] = jnp.zeros((B, 2 * HID), jnp.float32)
    c1_ref[...] = jnp.zeros((B, HID), jnp.float32)
    c2_ref[...] = jnp.zeros((B, HID), jnp.float32)

    def gates(g, c):
        i = jax.nn.sigmoid(g[:, 0:HID])
        f = jax.nn.sigmoid(g[:, HID:2 * HID])
        gg = jnp.tanh(g[:, 2 * HID:3 * HID])
        o = jax.nn.sigmoid(g[:, 3 * HID:4 * HID])
        c_new = f * c + i * gg
        return o * jnp.tanh(c_new), c_new

    def step(t, _):
        g1 = xi0_ref[pl.ds(t * B, B), :] + _mm(hcat_ref[:, :HID],
                                               whh0T_ref[...])
        h1, c1 = gates(g1, c1_ref[...])
        c1_ref[...] = c1
        hcat_ref[:, :HID] = h1

        g2 = _mm(hcat_ref[...], wcat1T_ref[...]) + b1_ref[...]
        h2, c2 = gates(g2, c2_ref[...])
        c2_ref[...] = c2
        hcat_ref[:, HID:] = h2
        y_ref[pl.ds(t * B, B), :] = h2
        return 0

    lax.fori_loop(0, S, step, 0)


def _lstm(x, wih0T, whh0T, wcat1T, b0, b1):
    return pl.pallas_call(
        _lstm_body,
        out_shape=jax.ShapeDtypeStruct((N_ROWS, HID), jnp.float32),
        scratch_shapes=[
            pltpu.VMEM((N_ROWS, G4), jnp.float32),
            pltpu.VMEM((B, 2 * HID), jnp.float32),
            pltpu.VMEM((B, HID), jnp.float32),
            pltpu.VMEM((B, HID), jnp.float32),
        ],
    )(x, wih0T, whh0T, wcat1T, b0, b1)


_HEAD_TILE = 128


def _head_body(y_ref, wg_hbm, bg_ref, out_ref, wg_f32, wg_bf, tile, sem, sem2):
    i = pl.program_id(0)

    @pl.when(i == 0)
    def _load_w():
        pltpu.make_async_copy(wg_hbm, wg_f32, sem).start()
        pltpu.make_async_copy(wg_hbm, wg_f32, sem).wait()
        wg_bf[...] = wg_f32[...].astype(jnp.bfloat16)

    logits = _mm(y_ref[...], wg_bf[...]) + bg_ref[...]
    m = jnp.max(logits, axis=1, keepdims=True)
    lse = jnp.log(jnp.sum(jnp.exp(logits - m), axis=1, keepdims=True)) + m
    tile[...] = logits - lse
    cp = pltpu.make_async_copy(
        tile, out_ref.at[pl.ds(i * _HEAD_TILE, _HEAD_TILE), :], sem2)
    cp.start()
    cp.wait()


def _head(y, wgT, bg):
    n_tiles = N_ROWS // _HEAD_TILE
    return pl.pallas_call(
        _head_body,
        grid=(n_tiles,),
        in_specs=[
            pl.BlockSpec((_HEAD_TILE, HID), lambda i: (i, 0)),
            pl.BlockSpec(memory_space=pl.ANY),
            pl.BlockSpec((1, VOCAB), lambda i: (0, 0)),
        ],
        out_specs=pl.BlockSpec(memory_space=pl.ANY),
        out_shape=jax.ShapeDtypeStruct((N_ROWS, VOCAB), jnp.float32),
        scratch_shapes=[
            pltpu.VMEM((HID, VOCAB), jnp.float32),
            pltpu.VMEM((HID, VOCAB), jnp.bfloat16),
            pltpu.VMEM((_HEAD_TILE, VOCAB), jnp.float32),
            pltpu.SemaphoreType.DMA,
            pltpu.SemaphoreType.DMA,
        ],
    )(y, wgT, bg)


def kernel(batchinput_tensor, embs_A, W_ih0, W_hh0, b_ih0, b_hh0,
           W_ih1, W_hh1, b_ih1, b_hh1, W_global, b_global):
    idx_t = batchinput_tensor[:, :, 0].astype(jnp.int32).T.reshape(N_ROWS)
    x = _make_sc_gather()(idx_t, embs_A)

    b0 = (b_ih0 + b_hh0).reshape(1, G4)
    b1 = (b_ih1 + b_hh1).reshape(1, G4)
    bf = jnp.bfloat16
    wih0T = W_ih0.T.astype(bf)
    whh0T = W_hh0.T.astype(bf)
    wcat1T = jnp.concatenate([W_ih1.T, W_hh1.T], axis=0).astype(bf)
    y_t = _lstm(x, wih0T, whh0T, wcat1T, b0, b1)

    task1 = y_t.reshape(S, B, HID).transpose(1, 0, 2).reshape(N_ROWS, HID)
    out = _head(task1, W_global.T, b_global.reshape(1, VOCAB))
    return (out, jnp.zeros((N_ROWS,), dtype=jnp.int32))


# speedup vs baseline: 1.2000x; 1.2000x over previous
import functools

import jax
import jax.numpy as jnp
from jax import lax
from jax.experimental import pallas as pl
from jax.experimental.pallas import tpu as pltpu
from jax.experimental.pallas import tpu_sc as plsc

B = 32
S = 32
DIM = 512
HID = 512
G4 = 4 * HID
N_ROWS = B * S
VOCAB = 10000

def _mm(x, w):
    return jnp.dot(x.astype(jnp.bfloat16), w,
                   preferred_element_type=jnp.float32)


@functools.lru_cache(maxsize=1)
def _make_sc_gather():
    info = plsc.get_sparse_core_info()
    nc, ns = info.num_cores, info.num_subcores
    nw = nc * ns
    rows_per_w = N_ROWS // nw
    mesh = plsc.VectorSubcoreMesh(core_axis_name="c", subcore_axis_name="s")

    @functools.partial(
        pl.kernel,
        mesh=mesh,
        out_type=jax.ShapeDtypeStruct((N_ROWS, DIM), jnp.float32),
        scratch_types=[
            pltpu.VMEM((rows_per_w,), jnp.int32),
            pltpu.VMEM((rows_per_w, DIM), jnp.float32),
            pltpu.SemaphoreType.DMA,
        ],
    )
    def gather_k(idx_hbm, table_hbm, out_hbm, idx_v, rows_v, sem):
        wid = lax.axis_index("s") * nc + lax.axis_index("c")
        base = wid * rows_per_w
        pltpu.sync_copy(idx_hbm.at[pl.ds(base, rows_per_w)], idx_v)
        pltpu.async_copy(table_hbm.at[idx_v], rows_v, sem).wait()
        pltpu.sync_copy(rows_v, out_hbm.at[pl.ds(base, rows_per_w)])

    return gather_k


def _lstm_body(x_ref, wih0_hbm, whh0_hbm, wih1_hbm, whh1_hbm, b0_ref, b1_ref,
               y_ref, xi0_ref, wih0T_ref, whh0T_ref, wcat1T_ref, wstage,
               hcat_ref, c1_ref, c2_ref, sem):
    def load_t(src_hbm, dst, row0):
        cp = pltpu.make_async_copy(src_hbm, wstage, sem)
        cp.start()
        cp.wait()
        for k in range(G4 // HID):
            dst[pl.ds(row0, HID), pl.ds(k * HID, HID)] = jnp.transpose(
                wstage[pl.ds(k * HID, HID), :], (1, 0)).astype(jnp.bfloat16)

    load_t(wih0_hbm, wih0T_ref, 0)
    load_t(whh0_hbm, whh0T_ref, 0)
    load_t(wih1_hbm, wcat1T_ref, 0)
    load_t(whh1_hbm, wcat1T_ref, HID)

    xi0_ref[...] = _mm(x_ref[...], wih0T_ref[...]) + b0_ref[...]
    hcat_ref[...] = jnp.zeros((B, 2 * HID), jnp.float32)
    c1_ref[...] = jnp.zeros((B, HID), jnp.float32)
    c2_ref[...] = jnp.zeros((B, HID), jnp.float32)

    def gates(g, c):
        i = jax.nn.sigmoid(g[:, 0:HID])
        f = jax.nn.sigmoid(g[:, HID:2 * HID])
        gg = jnp.tanh(g[:, 2 * HID:3 * HID])
        o = jax.nn.sigmoid(g[:, 3 * HID:4 * HID])
        c_new = f * c + i * gg
        return o * jnp.tanh(c_new), c_new

    def step(t, _):
        g1 = xi0_ref[pl.ds(t * B, B), :] + _mm(hcat_ref[:, :HID],
                                               whh0T_ref[...])
        h1, c1 = gates(g1, c1_ref[...])
        c1_ref[...] = c1
        hcat_ref[:, :HID] = h1

        g2 = _mm(hcat_ref[...], wcat1T_ref[...]) + b1_ref[...]
        h2, c2 = gates(g2, c2_ref[...])
        c2_ref[...] = c2
        hcat_ref[:, HID:] = h2
        y_ref[pl.ds(t * B, B), :] = h2
        return 0

    lax.fori_loop(0, S, step, 0)


def _lstm(x, wih0, whh0, wih1, whh1, b0, b1):
    return pl.pallas_call(
        _lstm_body,
        in_specs=[
            pl.BlockSpec((N_ROWS, DIM), lambda: (0, 0)),
            pl.BlockSpec(memory_space=pl.ANY),
            pl.BlockSpec(memory_space=pl.ANY),
            pl.BlockSpec(memory_space=pl.ANY),
            pl.BlockSpec(memory_space=pl.ANY),
            pl.BlockSpec((1, G4), lambda: (0, 0)),
            pl.BlockSpec((1, G4), lambda: (0, 0)),
        ],
        out_shape=jax.ShapeDtypeStruct((N_ROWS, HID), jnp.float32),
        scratch_shapes=[
            pltpu.VMEM((N_ROWS, G4), jnp.float32),
            pltpu.VMEM((HID, G4), jnp.bfloat16),
            pltpu.VMEM((HID, G4), jnp.bfloat16),
            pltpu.VMEM((2 * HID, G4), jnp.bfloat16),
            pltpu.VMEM((G4, DIM), jnp.float32),
            pltpu.VMEM((B, 2 * HID), jnp.float32),
            pltpu.VMEM((B, HID), jnp.float32),
            pltpu.VMEM((B, HID), jnp.float32),
            pltpu.SemaphoreType.DMA,
        ],
    )(x, wih0, whh0, wih1, whh1, b0, b1)


_HEAD_TILE = 128


def _head_body(y_ref, wg_hbm, bg_ref, out_ref, wg_f32, wg_bf, sem):
    @pl.when(pl.program_id(0) == 0)
    def _load_w():
        pltpu.make_async_copy(wg_hbm, wg_f32, sem).start()
        pltpu.make_async_copy(wg_hbm, wg_f32, sem).wait()
        wg_bf[...] = wg_f32[...].astype(jnp.bfloat16)

    logits = _mm(y_ref[...], wg_bf[...]) + bg_ref[...]
    m = jnp.max(logits, axis=1, keepdims=True)
    lse = jnp.log(jnp.sum(jnp.exp(logits - m), axis=1, keepdims=True)) + m
    out_ref[...] = (logits - lse).astype(jnp.bfloat16)


def _head(y, wgT, bg):
    n_tiles = N_ROWS // _HEAD_TILE
    return pl.pallas_call(
        _head_body,
        grid=(n_tiles,),
        in_specs=[
            pl.BlockSpec((_HEAD_TILE, HID), lambda i: (i, 0)),
            pl.BlockSpec(memory_space=pl.ANY),
            pl.BlockSpec((1, VOCAB), lambda i: (0, 0)),
        ],
        out_specs=pl.BlockSpec((_HEAD_TILE, VOCAB), lambda i: (i, 0)),
        out_shape=jax.ShapeDtypeStruct((N_ROWS, VOCAB), jnp.bfloat16),
        scratch_shapes=[
            pltpu.VMEM((HID, VOCAB), jnp.float32),
            pltpu.VMEM((HID, VOCAB), jnp.bfloat16),
            pltpu.SemaphoreType.DMA,
        ],
    )(y, wgT, bg)


def kernel(batchinput_tensor, embs_A, W_ih0, W_hh0, b_ih0, b_hh0,
           W_ih1, W_hh1, b_ih1, b_hh1, W_global, b_global):
    idx_t = batchinput_tensor[:, :, 0].astype(jnp.int32).T.reshape(N_ROWS)
    x = _make_sc_gather()(idx_t, embs_A)

    b0 = (b_ih0 + b_hh0).reshape(1, G4)
    b1 = (b_ih1 + b_hh1).reshape(1, G4)
    y_t = _lstm(x, W_ih0, W_hh0, W_ih1, W_hh1, b0, b1)

    task1 = y_t.reshape(S, B, HID).transpose(1, 0, 2).reshape(N_ROWS, HID)
    out = _head(task1, W_global.T, b_global.reshape(1, VOCAB))
    return (out.astype(jnp.float32), jnp.zeros((N_ROWS,), dtype=jnp.int32))


# speedup vs baseline: 1.2273x; 1.0227x over previous
import functools

import jax
import jax.numpy as jnp
from jax import lax
from jax.experimental import pallas as pl
from jax.experimental.pallas import tpu as pltpu
from jax.experimental.pallas import tpu_sc as plsc

B = 32
S = 32
DIM = 512
HID = 512
G4 = 4 * HID
N_ROWS = B * S
VOCAB = 10000

def _mm(x, w):
    return jnp.dot(x.astype(jnp.bfloat16), w,
                   preferred_element_type=jnp.float32)


@functools.lru_cache(maxsize=1)
def _make_sc_gather():
    info = plsc.get_sparse_core_info()
    nc, ns = info.num_cores, info.num_subcores
    nw = nc * ns
    rows_per_w = N_ROWS // nw
    mesh = plsc.VectorSubcoreMesh(core_axis_name="c", subcore_axis_name="s")

    @functools.partial(
        pl.kernel,
        mesh=mesh,
        out_type=jax.ShapeDtypeStruct((N_ROWS, DIM), jnp.float32),
        scratch_types=[
            pltpu.VMEM((rows_per_w,), jnp.int32),
            pltpu.VMEM((rows_per_w, DIM), jnp.float32),
            pltpu.SemaphoreType.DMA,
        ],
    )
    def gather_k(idx_hbm, table_hbm, out_hbm, idx_v, rows_v, sem):
        wid = lax.axis_index("s") * nc + lax.axis_index("c")
        base = wid * rows_per_w
        pltpu.sync_copy(idx_hbm.at[pl.ds(base, rows_per_w)], idx_v)
        pltpu.async_copy(table_hbm.at[idx_v], rows_v, sem).wait()
        pltpu.sync_copy(rows_v, out_hbm.at[pl.ds(base, rows_per_w)])

    return gather_k


def _lstm_body(x_ref, wih0_hbm, whh0_hbm, wih1_hbm, whh1_hbm, b0_ref, b1_ref,
               y_ref, xi0_ref, wih0T_ref, whh0T_ref, wcat1T_ref, wstage,
               hcat_ref, c1_ref, c2_ref, sem):
    def load_t(src_hbm, dst, row0):
        cp = pltpu.make_async_copy(src_hbm, wstage, sem)
        cp.start()
        cp.wait()
        for k in range(G4 // HID):
            dst[pl.ds(row0, HID), pl.ds(k * HID, HID)] = jnp.transpose(
                wstage[pl.ds(k * HID, HID), :], (1, 0)).astype(jnp.bfloat16)

    load_t(wih0_hbm, wih0T_ref, 0)
    load_t(whh0_hbm, whh0T_ref, 0)
    load_t(wih1_hbm, wcat1T_ref, 0)
    load_t(whh1_hbm, wcat1T_ref, HID)

    xi0_ref[...] = _mm(x_ref[...], wih0T_ref[...]) + b0_ref[...]
    hcat_ref[...] = jnp.zeros((B, 2 * HID), jnp.bfloat16)
    c1_ref[...] = jnp.zeros((B, HID), jnp.float32)
    c2_ref[...] = jnp.zeros((B, HID), jnp.float32)

    def gates(g, c):
        i = jax.nn.sigmoid(g[:, 0:HID])
        f = jax.nn.sigmoid(g[:, HID:2 * HID])
        gg = jnp.tanh(g[:, 2 * HID:3 * HID])
        o = jax.nn.sigmoid(g[:, 3 * HID:4 * HID])
        c_new = f * c + i * gg
        return o * jnp.tanh(c_new), c_new

    for t in range(S):
        g1 = xi0_ref[pl.ds(t * B, B), :] + _mm(hcat_ref[:, :HID],
                                               whh0T_ref[...])
        h1, c1 = gates(g1, c1_ref[...])
        c1_ref[...] = c1
        hcat_ref[:, :HID] = h1.astype(jnp.bfloat16)

        g2 = _mm(hcat_ref[...], wcat1T_ref[...]) + b1_ref[...]
        h2, c2 = gates(g2, c2_ref[...])
        c2_ref[...] = c2
        hcat_ref[:, HID:] = h2.astype(jnp.bfloat16)
        y_ref[pl.ds(t * B, B), :] = h2


def _lstm(x, wih0, whh0, wih1, whh1, b0, b1):
    return pl.pallas_call(
        _lstm_body,
        in_specs=[
            pl.BlockSpec((N_ROWS, DIM), lambda: (0, 0)),
            pl.BlockSpec(memory_space=pl.ANY),
            pl.BlockSpec(memory_space=pl.ANY),
            pl.BlockSpec(memory_space=pl.ANY),
            pl.BlockSpec(memory_space=pl.ANY),
            pl.BlockSpec((1, G4), lambda: (0, 0)),
            pl.BlockSpec((1, G4), lambda: (0, 0)),
        ],
        out_shape=jax.ShapeDtypeStruct((N_ROWS, HID), jnp.float32),
        scratch_shapes=[
            pltpu.VMEM((N_ROWS, G4), jnp.float32),
            pltpu.VMEM((HID, G4), jnp.bfloat16),
            pltpu.VMEM((HID, G4), jnp.bfloat16),
            pltpu.VMEM((2 * HID, G4), jnp.bfloat16),
            pltpu.VMEM((G4, DIM), jnp.float32),
            pltpu.VMEM((B, 2 * HID), jnp.bfloat16),
            pltpu.VMEM((B, HID), jnp.float32),
            pltpu.VMEM((B, HID), jnp.float32),
            pltpu.SemaphoreType.DMA,
        ],
    )(x, wih0, whh0, wih1, whh1, b0, b1)


_HEAD_TILE = 128


def _head_body(y_ref, wg_hbm, bg_ref, out_ref, wg_f32, wg_bf, sem):
    @pl.when(pl.program_id(0) == 0)
    def _load_w():
        pltpu.make_async_copy(wg_hbm, wg_f32, sem).start()
        pltpu.make_async_copy(wg_hbm, wg_f32, sem).wait()
        wg_bf[...] = wg_f32[...].astype(jnp.bfloat16)

    logits = _mm(y_ref[...], wg_bf[...]) + bg_ref[...]
    m = jnp.max(logits, axis=1, keepdims=True)
    lse = jnp.log(jnp.sum(jnp.exp(logits - m), axis=1, keepdims=True)) + m
    out_ref[...] = (logits - lse).astype(jnp.bfloat16)


def _head(y, wgT, bg):
    n_tiles = N_ROWS // _HEAD_TILE
    return pl.pallas_call(
        _head_body,
        grid=(n_tiles,),
        in_specs=[
            pl.BlockSpec((_HEAD_TILE, HID), lambda i: (i, 0)),
            pl.BlockSpec(memory_space=pl.ANY),
            pl.BlockSpec((1, VOCAB), lambda i: (0, 0)),
        ],
        out_specs=pl.BlockSpec((_HEAD_TILE, VOCAB), lambda i: (i, 0)),
        out_shape=jax.ShapeDtypeStruct((N_ROWS, VOCAB), jnp.bfloat16),
        scratch_shapes=[
            pltpu.VMEM((HID, VOCAB), jnp.float32),
            pltpu.VMEM((HID, VOCAB), jnp.bfloat16),
            pltpu.SemaphoreType.DMA,
        ],
    )(y, wgT, bg)


def kernel(batchinput_tensor, embs_A, W_ih0, W_hh0, b_ih0, b_hh0,
           W_ih1, W_hh1, b_ih1, b_hh1, W_global, b_global):
    idx_t = batchinput_tensor[:, :, 0].astype(jnp.int32).T.reshape(N_ROWS)
    x = _make_sc_gather()(idx_t, embs_A)

    b0 = (b_ih0 + b_hh0).reshape(1, G4)
    b1 = (b_ih1 + b_hh1).reshape(1, G4)
    y_t = _lstm(x, W_ih0, W_hh0, W_ih1, W_hh1, b0, b1)

    task1 = y_t.reshape(S, B, HID).transpose(1, 0, 2).reshape(N_ROWS, HID)
    out = _head(task1, W_global.T, b_global.reshape(1, VOCAB))
    return (out.astype(jnp.float32), jnp.zeros((N_ROWS,), dtype=jnp.int32))


# speedup vs baseline: 1.2424x; 1.0123x over previous
import functools

import jax
import jax.numpy as jnp
from jax import lax
from jax.experimental import pallas as pl
from jax.experimental.pallas import tpu as pltpu
from jax.experimental.pallas import tpu_sc as plsc

B = 32
S = 32
DIM = 512
HID = 512
G4 = 4 * HID
N_ROWS = B * S
VOCAB = 10000

def _mm(x, w):
    return jnp.dot(x.astype(jnp.bfloat16), w,
                   preferred_element_type=jnp.float32)


@functools.lru_cache(maxsize=1)
def _make_sc_gather():
    info = plsc.get_sparse_core_info()
    nc, ns = info.num_cores, info.num_subcores
    nw = nc * ns
    rows_per_w = N_ROWS // nw
    mesh = plsc.VectorSubcoreMesh(core_axis_name="c", subcore_axis_name="s")

    @functools.partial(
        pl.kernel,
        mesh=mesh,
        out_type=jax.ShapeDtypeStruct((N_ROWS, DIM), jnp.float32),
        scratch_types=[
            pltpu.VMEM((rows_per_w,), jnp.int32),
            pltpu.VMEM((rows_per_w, DIM), jnp.float32),
            pltpu.SemaphoreType.DMA,
        ],
    )
    def gather_k(idx_hbm, table_hbm, out_hbm, idx_v, rows_v, sem):
        wid = lax.axis_index("s") * nc + lax.axis_index("c")
        base = wid * rows_per_w
        pltpu.sync_copy(idx_hbm.at[pl.ds(base, rows_per_w)], idx_v)
        pltpu.async_copy(table_hbm.at[idx_v], rows_v, sem).wait()
        pltpu.sync_copy(rows_v, out_hbm.at[pl.ds(base, rows_per_w)])

    return gather_k


def _lstm_body(x_ref, wih0_hbm, whh0_hbm, wih1_hbm, whh1_hbm, b0_ref, b1_ref,
               y_ref, xi0_ref, wih0T_ref, whh0T_ref, wcat1T_ref, wstage,
               hcat_ref, c1_ref, c2_ref, sem):
    def load_t(src_hbm, dst, row0):
        cp = pltpu.make_async_copy(src_hbm, wstage, sem)
        cp.start()
        cp.wait()
        for k in range(G4 // HID):
            dst[pl.ds(row0, HID), pl.ds(k * HID, HID)] = jnp.transpose(
                wstage[pl.ds(k * HID, HID), :], (1, 0)).astype(jnp.bfloat16)

    load_t(wih0_hbm, wih0T_ref, 0)
    load_t(whh0_hbm, whh0T_ref, 0)
    load_t(wih1_hbm, wcat1T_ref, 0)
    load_t(whh1_hbm, wcat1T_ref, HID)

    xi0_ref[...] = _mm(x_ref[...], wih0T_ref[...]) + b0_ref[...]
    hcat_ref[...] = jnp.zeros((B, 2 * HID), jnp.bfloat16)
    c1_ref[...] = jnp.zeros((B, HID), jnp.float32)
    c2_ref[...] = jnp.zeros((B, HID), jnp.float32)

    def gates(g, c):
        i = jax.nn.sigmoid(g[:, 0:HID])
        f = jax.nn.sigmoid(g[:, HID:2 * HID])
        gg = jnp.tanh(g[:, 2 * HID:3 * HID])
        o = jax.nn.sigmoid(g[:, 3 * HID:4 * HID])
        c_new = f * c + i * gg
        return o * jnp.tanh(c_new), c_new

    for t in range(S):
        g1 = xi0_ref[pl.ds(t * B, B), :] + _mm(hcat_ref[:, :HID],
                                               whh0T_ref[...])
        h1, c1 = gates(g1, c1_ref[...])
        c1_ref[...] = c1
        hcat_ref[:, :HID] = h1.astype(jnp.bfloat16)

        g2 = _mm(hcat_ref[...], wcat1T_ref[...]) + b1_ref[...]
        h2, c2 = gates(g2, c2_ref[...])
        c2_ref[...] = c2
        hcat_ref[:, HID:] = h2.astype(jnp.bfloat16)
        y_ref[pl.ds(t * B, B), :] = h2


def _lstm(x, wih0, whh0, wih1, whh1, b0, b1):
    return pl.pallas_call(
        _lstm_body,
        in_specs=[
            pl.BlockSpec((N_ROWS, DIM), lambda: (0, 0)),
            pl.BlockSpec(memory_space=pl.ANY),
            pl.BlockSpec(memory_space=pl.ANY),
            pl.BlockSpec(memory_space=pl.ANY),
            pl.BlockSpec(memory_space=pl.ANY),
            pl.BlockSpec((1, G4), lambda: (0, 0)),
            pl.BlockSpec((1, G4), lambda: (0, 0)),
        ],
        out_shape=jax.ShapeDtypeStruct((N_ROWS, HID), jnp.float32),
        scratch_shapes=[
            pltpu.VMEM((N_ROWS, G4), jnp.float32),
            pltpu.VMEM((HID, G4), jnp.bfloat16),
            pltpu.VMEM((HID, G4), jnp.bfloat16),
            pltpu.VMEM((2 * HID, G4), jnp.bfloat16),
            pltpu.VMEM((G4, DIM), jnp.float32),
            pltpu.VMEM((B, 2 * HID), jnp.bfloat16),
            pltpu.VMEM((B, HID), jnp.float32),
            pltpu.VMEM((B, HID), jnp.float32),
            pltpu.SemaphoreType.DMA,
        ],
    )(x, wih0, whh0, wih1, whh1, b0, b1)


_HEAD_TILE = 256


def _head_body(y_ref, wg_hbm, bg_ref, out_ref, wg_f32, wg_bf, sem):
    @pl.when(pl.program_id(0) == 0)
    def _load_w():
        pltpu.make_async_copy(wg_hbm, wg_f32, sem).start()
        pltpu.make_async_copy(wg_hbm, wg_f32, sem).wait()
        wg_bf[...] = wg_f32[...].astype(jnp.bfloat16)

    yb = jnp.transpose(y_ref[...], (1, 0, 2)).reshape(_HEAD_TILE, HID)
    logits = _mm(yb, wg_bf[...]) + bg_ref[...]
    m = jnp.max(logits, axis=1, keepdims=True)
    lse = jnp.log(jnp.sum(jnp.exp(logits - m), axis=1, keepdims=True)) + m
    out_ref[...] = (logits - lse).astype(jnp.bfloat16)


def _head(y, wgT, bg):
    n_tiles = N_ROWS // _HEAD_TILE
    return pl.pallas_call(
        _head_body,
        grid=(n_tiles,),
        in_specs=[
            pl.BlockSpec((S, _HEAD_TILE // S, HID), lambda i: (0, i, 0)),
            pl.BlockSpec(memory_space=pl.ANY),
            pl.BlockSpec((1, VOCAB), lambda i: (0, 0)),
        ],
        out_specs=pl.BlockSpec((_HEAD_TILE, VOCAB), lambda i: (i, 0)),
        out_shape=jax.ShapeDtypeStruct((N_ROWS, VOCAB), jnp.bfloat16),
        scratch_shapes=[
            pltpu.VMEM((HID, VOCAB), jnp.float32),
            pltpu.VMEM((HID, VOCAB), jnp.bfloat16),
            pltpu.SemaphoreType.DMA,
        ],
    )(y, wgT, bg)


def kernel(batchinput_tensor, embs_A, W_ih0, W_hh0, b_ih0, b_hh0,
           W_ih1, W_hh1, b_ih1, b_hh1, W_global, b_global):
    idx_t = batchinput_tensor[:, :, 0].astype(jnp.int32).T.reshape(N_ROWS)
    x = _make_sc_gather()(idx_t, embs_A)

    b0 = (b_ih0 + b_hh0).reshape(1, G4)
    b1 = (b_ih1 + b_hh1).reshape(1, G4)
    y_t = _lstm(x, W_ih0, W_hh0, W_ih1, W_hh1, b0, b1)

    out = _head(y_t.reshape(S, B, HID), W_global.T, b_global.reshape(1, VOCAB))
    return (out.astype(jnp.float32), jnp.zeros((N_ROWS,), dtype=jnp.int32))


# speedup vs baseline: 1.2674x; 1.0201x over previous
import functools

import jax
import jax.numpy as jnp
from jax import lax
from jax.experimental import pallas as pl
from jax.experimental.pallas import tpu as pltpu
from jax.experimental.pallas import tpu_sc as plsc

B = 32
S = 32
DIM = 512
HID = 512
G4 = 4 * HID
N_ROWS = B * S
VOCAB = 10000

def _mm(x, w):
    return jnp.dot(x.astype(jnp.bfloat16), w,
                   preferred_element_type=jnp.float32)


@functools.lru_cache(maxsize=1)
def _make_sc_gather():
    info = plsc.get_sparse_core_info()
    nc, ns = info.num_cores, info.num_subcores
    nw = nc * ns
    rows_per_w = N_ROWS // nw
    mesh = plsc.VectorSubcoreMesh(core_axis_name="c", subcore_axis_name="s")

    @functools.partial(
        pl.kernel,
        mesh=mesh,
        out_type=jax.ShapeDtypeStruct((N_ROWS, DIM), jnp.float32),
        scratch_types=[
            pltpu.VMEM((rows_per_w,), jnp.int32),
            pltpu.VMEM((rows_per_w, DIM), jnp.float32),
            pltpu.SemaphoreType.DMA,
        ],
    )
    def gather_k(idx_hbm, table_hbm, out_hbm, idx_v, rows_v, sem):
        wid = lax.axis_index("s") * nc + lax.axis_index("c")
        base = wid * rows_per_w
        pltpu.sync_copy(idx_hbm.at[pl.ds(base, rows_per_w)], idx_v)
        pltpu.async_copy(table_hbm.at[idx_v], rows_v, sem).wait()
        pltpu.sync_copy(rows_v, out_hbm.at[pl.ds(base, rows_per_w)])

    return gather_k


def _lstm_body(x_ref, wih0_hbm, whh0_hbm, wih1_hbm, whh1_hbm, b0_ref, b1_ref,
               y_ref, xi0_ref, wih0T_ref, whh0T_ref, wcat1T_ref, wstage,
               hcat_ref, c1_ref, c2_ref, sem):
    def load_t(src_hbm, dst, row0):
        cp = pltpu.make_async_copy(src_hbm, wstage, sem)
        cp.start()
        cp.wait()
        for k in range(G4 // HID):
            dst[pl.ds(row0, HID), pl.ds(k * HID, HID)] = jnp.transpose(
                wstage[pl.ds(k * HID, HID), :], (1, 0)).astype(jnp.bfloat16)

    load_t(wih0_hbm, wih0T_ref, 0)

    xi0_ref[...] = _mm(x_ref[...], wih0T_ref[...]) + b0_ref[...]

    load_t(whh0_hbm, whh0T_ref, 0)
    load_t(wih1_hbm, wcat1T_ref, 0)
    load_t(whh1_hbm, wcat1T_ref, HID)
    hcat_ref[...] = jnp.zeros((B, 2 * HID), jnp.bfloat16)
    c1_ref[...] = jnp.zeros((B, HID), jnp.float32)
    c2_ref[...] = jnp.zeros((B, HID), jnp.float32)

    def gates(g, c):
        i = jax.nn.sigmoid(g[:, 0:HID])
        f = jax.nn.sigmoid(g[:, HID:2 * HID])
        gg = jnp.tanh(g[:, 2 * HID:3 * HID])
        o = jax.nn.sigmoid(g[:, 3 * HID:4 * HID])
        c_new = f * c + i * gg
        return o * jnp.tanh(c_new), c_new

    for t in range(S):
        g1 = xi0_ref[pl.ds(t * B, B), :] + _mm(hcat_ref[:, :HID],
                                               whh0T_ref[...])
        h1, c1 = gates(g1, c1_ref[...])
        c1_ref[...] = c1
        hcat_ref[:, :HID] = h1.astype(jnp.bfloat16)

        g2 = _mm(hcat_ref[...], wcat1T_ref[...]) + b1_ref[...]
        h2, c2 = gates(g2, c2_ref[...])
        c2_ref[...] = c2
        hcat_ref[:, HID:] = h2.astype(jnp.bfloat16)
        y_ref[pl.ds(t * B, B), :] = h2


def _lstm(x, wih0, whh0, wih1, whh1, b0, b1):
    return pl.pallas_call(
        _lstm_body,
        in_specs=[
            pl.BlockSpec((N_ROWS, DIM), lambda: (0, 0)),
            pl.BlockSpec(memory_space=pl.ANY),
            pl.BlockSpec(memory_space=pl.ANY),
            pl.BlockSpec(memory_space=pl.ANY),
            pl.BlockSpec(memory_space=pl.ANY),
            pl.BlockSpec((1, G4), lambda: (0, 0)),
            pl.BlockSpec((1, G4), lambda: (0, 0)),
        ],
        out_shape=jax.ShapeDtypeStruct((N_ROWS, HID), jnp.float32),
        scratch_shapes=[
            pltpu.VMEM((N_ROWS, G4), jnp.float32),
            pltpu.VMEM((HID, G4), jnp.bfloat16),
            pltpu.VMEM((HID, G4), jnp.bfloat16),
            pltpu.VMEM((2 * HID, G4), jnp.bfloat16),
            pltpu.VMEM((G4, DIM), jnp.float32),
            pltpu.VMEM((B, 2 * HID), jnp.bfloat16),
            pltpu.VMEM((B, HID), jnp.float32),
            pltpu.VMEM((B, HID), jnp.float32),
            pltpu.SemaphoreType.DMA,
        ],
    )(x, wih0, whh0, wih1, whh1, b0, b1)


_HEAD_TILE = 256


def _head_body(y_ref, wg_hbm, bg_ref, out_ref, wg_f32, wg_bf, sem):
    @pl.when(pl.program_id(0) == 0)
    def _load_w():
        pltpu.make_async_copy(wg_hbm, wg_f32, sem).start()
        pltpu.make_async_copy(wg_hbm, wg_f32, sem).wait()
        wg_bf[...] = wg_f32[...].astype(jnp.bfloat16)

    yb = jnp.transpose(y_ref[...], (1, 0, 2)).reshape(_HEAD_TILE, HID)
    logits = _mm(yb, wg_bf[...]) + bg_ref[...]
    m = jnp.max(logits, axis=1, keepdims=True)
    lse = jnp.log(jnp.sum(jnp.exp(logits - m), axis=1, keepdims=True)) + m
    out_ref[...] = (logits - lse).astype(jnp.bfloat16)


def _head(y, wgT, bg):
    n_tiles = N_ROWS // _HEAD_TILE
    return pl.pallas_call(
        _head_body,
        grid=(n_tiles,),
        in_specs=[
            pl.BlockSpec((S, _HEAD_TILE // S, HID), lambda i: (0, i, 0)),
            pl.BlockSpec(memory_space=pl.ANY),
            pl.BlockSpec((1, VOCAB), lambda i: (0, 0)),
        ],
        out_specs=pl.BlockSpec((_HEAD_TILE, VOCAB), lambda i: (i, 0)),
        out_shape=jax.ShapeDtypeStruct((N_ROWS, VOCAB), jnp.bfloat16),
        scratch_shapes=[
            pltpu.VMEM((HID, VOCAB), jnp.float32),
            pltpu.VMEM((HID, VOCAB), jnp.bfloat16),
            pltpu.SemaphoreType.DMA,
        ],
    )(y, wgT, bg)


def kernel(batchinput_tensor, embs_A, W_ih0, W_hh0, b_ih0, b_hh0,
           W_ih1, W_hh1, b_ih1, b_hh1, W_global, b_global):
    idx_t = batchinput_tensor[:, :, 0].astype(jnp.int32).T.reshape(N_ROWS)
    x = _make_sc_gather()(idx_t, embs_A)

    b0 = (b_ih0 + b_hh0).reshape(1, G4)
    b1 = (b_ih1 + b_hh1).reshape(1, G4)
    y_t = _lstm(x, W_ih0, W_hh0, W_ih1, W_hh1, b0, b1)

    out = _head(y_t.reshape(S, B, HID), W_global.T, b_global.reshape(1, VOCAB))
    return (out.astype(jnp.float32), jnp.zeros((N_ROWS,), dtype=jnp.int32))
